# baseline (speedup 1.0000x reference)
import math

import jax
import jax.numpy as jnp
import numpy as np
from jax.experimental import pallas as pl
from jax.experimental.pallas import tpu as pltpu

_ORDER = 13
_SEED = 48879

_CHUNK = 1024
_NBUF = 4
_MBUF = 4


def _hilbert_encode_vec(x, y, order):
    x = x.astype(np.int64).copy()
    y = y.astype(np.int64).copy()
    d = np.zeros_like(x)
    s = 1 << (order - 1)
    while s > 0:
        rx = ((x & s) > 0).astype(np.int64)
        ry = ((y & s) > 0).astype(np.int64)
        d += s * s * ((3 * rx) ^ ry)
        swap = ry == 0
        flip = swap & (rx == 1)
        x_f = np.where(flip, s - 1 - x, x)
        y_f = np.where(flip, s - 1 - y, y)
        x_new = np.where(swap, y_f, x_f)
        y_new = np.where(swap, x_f, y_f)
        x, y = x_new, y_new
        s >>= 1
    return d


def _v_mask_generative(addr_u64, rounds, seed):
    h = addr_u64 ^ np.uint64(seed & 0xFFFFFFFFFFFFFFFF)
    for _ in range(rounds):
        h = h * np.uint64(6364136223846793005) + np.uint64(1442695040888963407)
        h = h ^ (h >> np.uint64(33))
    frac = (h & np.uint64(0xFFFFFF)).astype(np.float64) / float(0xFFFFFF)
    return (frac * 2.0 - 1.0).astype(np.float32)


def _resonance_vec(S, D, seed_val, order):
    i = np.arange(S, dtype=np.int64)
    j = i.copy()
    t = _hilbert_encode_vec(i, j, order)
    addr = (i.astype(np.uint64) << np.uint64(32)) | j.astype(np.uint64)
    s_long = int(round(seed_val))
    ground_weight = _v_mask_generative(addr, 4, s_long ^ D)
    sig = (np.uint64(s_long) ^ np.uint64(D) ^ t.astype(np.uint64)) & np.uint64(0xFFFFFFFF)
    phase = (sig % np.uint64(1000)).astype(np.float64) / 1000.0 * 2.0 * math.pi
    resonance = np.sin(phase).astype(np.float32)
    return ground_weight * resonance


def _make_pipe_body(n_steps):
    def _pipe_body(scale_ref, res_ref, q_hbm, o_hbm, inbuf, outbuf, in_sems, out_sems):
        s = scale_ref[0, 0]

        def in_copy(i, slot):
            return pltpu.make_async_copy(
                q_hbm.at[pl.ds(i * _CHUNK, _CHUNK), :], inbuf.at[slot], in_sems.at[slot]
            )

        def out_copy(i, slot):
            return pltpu.make_async_copy(
                outbuf.at[slot], o_hbm.at[pl.ds(i * _CHUNK, _CHUNK), :], out_sems.at[slot]
            )

        for j in range(min(_NBUF, n_steps)):
            in_copy(j, j).start()

        for i in range(n_steps):
            islot = i % _NBUF
            oslot = i % _MBUF
            if i >= _MBUF:
                out_copy(i - _MBUF, oslot).wait()
            in_copy(i, islot).wait()
            r = res_ref[pl.ds(i * _CHUNK, _CHUNK), :]
            outbuf[oslot] = inbuf[islot] * (r * s)
            nxt = i + _NBUF
            if nxt < n_steps:
                in_copy(nxt, islot).start()
            out_copy(i, oslot).start()

        for i in range(max(0, n_steps - _MBUF), n_steps):
            out_copy(i, i % _MBUF).wait()

    return _pipe_body


def kernel(q, k, v_val, latent_seed):
    B, S, D = q.shape
    res = _resonance_vec(S, D, float(_SEED), _ORDER)
    res_full = jnp.asarray(np.tile(res, B).reshape(B * S, 1))
    scale = (latent_seed * jnp.float32(1.0 / _SEED)).reshape(1, 1)

    rows = B * S
    n_steps = rows // _CHUNK
    q2 = q.reshape(rows, D)

    out = pl.pallas_call(
        _make_pipe_body(n_steps),
        in_specs=[
            pl.BlockSpec(memory_space=pltpu.SMEM),
            pl.BlockSpec(memory_space=pltpu.VMEM),
            pl.BlockSpec(memory_space=pltpu.HBM),
        ],
        out_specs=pl.BlockSpec(memory_space=pltpu.HBM),
        out_shape=jax.ShapeDtypeStruct((rows, D), jnp.float32),
        scratch_shapes=[
            pltpu.VMEM((_NBUF, _CHUNK, D), jnp.float32),
            pltpu.VMEM((_MBUF, _CHUNK, D), jnp.float32),
            pltpu.SemaphoreType.DMA((_NBUF,)),
            pltpu.SemaphoreType.DMA((_MBUF,)),
        ],
    )(scale, res_full, q2)
    return out.reshape(B, S, D)
